# baseline (speedup 1.0000x reference)
import functools

import jax
import jax.numpy as jnp
from jax import lax
from jax.experimental import pallas as pl
from jax.experimental.pallas import tpu as pltpu
from jax.experimental.pallas import tpu_sc as plsc

B, C, H, W = 8, 3, 512, 512
HW = H * W
NW = 32
SHARD = HW // NW
NPC = 64
PCPTS = (B * HW) // NPC
PCB = NPC // B
ACH = 2048
NCH = PCPTS // ACH
CAP = 768
NSEG = NPC
SEGB = PCB
ROWS = 128
MAXP = HW


def _prep_body(flow_ref, depth_ref, lin_ref, pvn_ref):
    r = pl.program_id(1)
    fx = flow_ref[0, 0]
    fy = flow_ref[0, 1]
    gx = lax.broadcasted_iota(jnp.int32, (ROWS, W), 1).astype(jnp.float32)
    gy = lax.broadcasted_iota(jnp.int32, (ROWS, W), 0).astype(jnp.float32) \
        + (r * ROWS).astype(jnp.float32)
    cxf = jnp.round(gx + fx)
    cyf = jnp.round(gy + fy)
    inb = (cxf >= 0) & (cxf < W) & (cyf >= 0) & (cyf < H)
    cx = jnp.clip(cxf, 0, W - 1).astype(jnp.int32)
    cy = jnp.clip(cyf, 0, H - 1).astype(jnp.int32)
    lin_ref[0] = jnp.where(inb, cy * W + cx, 0)
    v = depth_ref[0, 0]
    pvn = 1.0 / (v + 1e-08)
    pvn_ref[0] = pvn * (pvn < 10000.0).astype(jnp.float32)


def _prep(x, flow_in):
    lin, pvn = pl.pallas_call(
        _prep_body,
        out_shape=(
            jax.ShapeDtypeStruct((B, H, W), jnp.int32),
            jax.ShapeDtypeStruct((B, H, W), jnp.float32),
        ),
        grid=(B, H // ROWS),
        in_specs=[
            pl.BlockSpec((1, 2, ROWS, W), lambda b, r: (b, 0, r, 0)),
            pl.BlockSpec((1, 1, ROWS, W), lambda b, r: (b, 2, r, 0)),
        ],
        out_specs=(
            pl.BlockSpec((1, ROWS, W), lambda b, r: (b, r, 0)),
            pl.BlockSpec((1, ROWS, W), lambda b, r: (b, r, 0)),
        ),
    )(flow_in, x)
    return lin.reshape(B * HW), pvn.reshape(B * HW)


def _bin_body(lin_hbm, pvn_hbm, pack_hbm, pvnb_hbm, cnt_hbm,
              lin_v, pvn_v, next_v, stage_pk, stage_pv, sem):
    wid = lax.axis_index("s") * 2 + lax.axis_index("c")
    iota = lax.iota(jnp.int32, 16)

    def per_pc(k, _):
        pc = wid * 2 + k
        pt0 = pc * PCPTS
        p_base = (pc % PCB) * PCPTS

        next_v[pl.ds(0, 16)] = jnp.zeros((16,), jnp.int32)
        next_v[pl.ds(16, 16)] = jnp.zeros((16,), jnp.int32)
        next_v[pl.ds(32, 16)] = jnp.zeros((16,), jnp.int32)
        next_v[pl.ds(48, 16)] = jnp.zeros((16,), jnp.int32)

        def per_chunk(ch, _):
            off = pt0 + ch * ACH
            pltpu.sync_copy(lin_hbm.at[pl.ds(off, ACH)], lin_v)
            pltpu.sync_copy(pvn_hbm.at[pl.ds(off, ACH)], pvn_v)

            def vloop(i, _):
                l = lin_v[pl.ds(i * 16, 16)]
                v = pvn_v[pl.ds(i * 16, 16)]
                act = v > 0.0
                bkt = lax.shift_right_logical(l, 4) & 31
                key = jnp.where(act, bkt, 63)
                p = p_base + ch * ACH + i * 16 + iota
                floc = (l & 15) * 512 + lax.shift_right_logical(l, 9)
                pack = lax.shift_left(floc, 18) | p
                skey, sval = plsc.sort_key_val(key, iota)
                v_s = jnp.take(v, sval)
                pack_s = jnp.take(pack, sval)
                act_s = skey < 32
                st = iota
                c = ((skey == jnp.take(skey, jnp.maximum(iota - 1, 0)))
                     & (iota >= 1)).astype(jnp.int32)
                for d in (1, 2, 4, 8):
                    back = jnp.maximum(iota - d, 0)
                    st = jnp.where(c != 0, jnp.take(st, back), st)
                    c = c & jnp.take(c, back)
                rank = iota - st
                nxt_key = jnp.take(skey, jnp.minimum(iota + 1, 15))
                is_last = (iota == 15) | (nxt_key != skey)
                cur = plsc.load_gather(next_v, [skey])
                pos = cur + rank
                valid = act_s & (pos < CAP)
                plsc.store_scatter(next_v, [skey], pos + 1,
                                   mask=is_last & act_s)
                sidx = skey * CAP + jnp.minimum(pos, CAP - 1)
                plsc.store_scatter(stage_pk, [sidx], pack_s, mask=valid)
                plsc.store_scatter(stage_pv, [sidx], v_s, mask=valid)
                return 0

            lax.fori_loop(0, ACH // 16, vloop, 0)
            return 0

        lax.fori_loop(0, NCH, per_chunk, 0)
        pltpu.sync_copy(stage_pk,
                        pack_hbm.at[pl.ds(pc * 32 * CAP, 32 * CAP)])
        pltpu.sync_copy(stage_pv,
                        pvnb_hbm.at[pl.ds(pc * 32 * CAP, 32 * CAP)])
        pltpu.sync_copy(next_v.at[pl.ds(0, 32)],
                        cnt_hbm.at[pl.ds(pc * 32, 32)])
        return 0

    lax.fori_loop(0, 2, per_pc, 0)


def _splat_body(pack_hbm, pvnb_hbm, cnt_hbm, x_hbm, lin_hbm, pvn_hbm, out_hbm,
                pkbuf, pvbuf, cntbuf, maxv_fb, argp_fb, idx_v, gath_v, outb_v,
                lin_v, pvn_v, sem, sem2):
    wid = lax.axis_index("s") * 2 + lax.axis_index("c")
    iota = lax.iota(jnp.int32, 16)

    def per_batch(b, _):
        pt_base = b * HW

        def init(i, _):
            maxv_fb[pl.ds(i * 16, 16)] = jnp.zeros((16,), jnp.float32)
            argp_fb[pl.ds(i * 16, 16)] = jnp.full((16,), MAXP, jnp.int32)
            return 0

        lax.fori_loop(0, SHARD // 16, init, 0)

        pltpu.sync_copy(cnt_hbm.at[pl.ds(b * SEGB * 32, SEGB * 32)], cntbuf)

        def fire(j, _):
            soff = ((b * SEGB + j) * 32 + wid) * CAP
            pltpu.async_copy(pack_hbm.at[pl.ds(soff, CAP)],
                             pkbuf.at[pl.ds(j * CAP, CAP)], sem)
            pltpu.async_copy(pvnb_hbm.at[pl.ds(soff, CAP)],
                             pvbuf.at[pl.ds(j * CAP, CAP)], sem2)
            return 0

        lax.fori_loop(0, SEGB, fire, 0)
        pltpu.make_async_copy(pack_hbm.at[pl.ds(0, SEGB * CAP)], pkbuf, sem).wait()
        pltpu.make_async_copy(pvnb_hbm.at[pl.ds(0, SEGB * CAP)], pvbuf, sem2).wait()

        def ovf_scan(j, m):
            cj = cntbuf[pl.ds(j * 16, 16)]
            return jnp.maximum(m, jnp.max(cj))

        max_cnt = lax.fori_loop(0, (SEGB * 32) // 16, ovf_scan, jnp.int32(0))

        def pass1_j(j, _):
            cseg = plsc.load_gather(cntbuf, [jnp.full((16,), j * 32 + wid,
                                                      jnp.int32)])
            trip = lax.div(jnp.max(jnp.minimum(cseg, CAP)) + 15, 16)
            bj = j * CAP

            def it(i, _):
                lane = i * 16 + iota
                pk = pkbuf[pl.ds(bj + i * 16, 16)]
                v = pvbuf[pl.ds(bj + i * 16, 16)]
                lm = lane < cseg
                spk, sv = plsc.sort_key_val(jnp.where(lm, pk, -1), v)
                am = spk >= 0
                floc = lax.shift_right_logical(spk, 18)
                prev = jnp.take(floc, jnp.maximum(iota - 1, 0))
                m = sv
                c = ((floc == prev) & (iota >= 1)).astype(jnp.int32)
                for d in (1, 2, 4, 8):
                    back = jnp.maximum(iota - d, 0)
                    m = jnp.where(c != 0, jnp.maximum(m, jnp.take(m, back)), m)
                    c = c & jnp.take(c, back)
                nxt = jnp.take(floc, jnp.minimum(iota + 1, 15))
                is_last = (iota == 15) | (nxt != floc)
                flocc = jnp.clip(floc, 0, SHARD - 1)
                g = plsc.load_gather(maxv_fb, [flocc])
                mm = am & is_last & (m > g)
                plsc.store_scatter(maxv_fb, [flocc], m, mask=mm)
                return 0

            lax.fori_loop(0, trip, it, 0)
            return 0

        lax.fori_loop(0, SEGB, pass1_j, 0)

        @pl.when(max_cnt > CAP)
        def _():
            def f1_chunk(ci, _):
                off = pt_base + ci * ACH
                pltpu.sync_copy(lin_hbm.at[pl.ds(off, ACH)], lin_v)
                pltpu.sync_copy(pvn_hbm.at[pl.ds(off, ACH)], pvn_v)

                def vloop(i, _):
                    l = lin_v[pl.ds(i * 16, 16)]
                    v = pvn_v[pl.ds(i * 16, 16)]
                    cs = lax.shift_right_logical(l, 4) & 31
                    act = (cs == wid) & (v > 0.0)
                    locs = (l & 15) * 512 + lax.shift_right_logical(l, 9)
                    g = plsc.load_gather(maxv_fb, [locs])
                    need = act & (v > g)

                    def body(m):
                        plsc.store_scatter(maxv_fb, [locs], v, mask=m != 0)
                        g2 = plsc.load_gather(maxv_fb, [locs])
                        return (act & (v > g2)).astype(jnp.int32)

                    lax.while_loop(lambda m: jnp.any(m != 0), body,
                                   need.astype(jnp.int32))
                    return 0

                lax.fori_loop(0, ACH // 16, vloop, 0)
                return 0

            lax.fori_loop(0, HW // ACH, f1_chunk, 0)

        def pass2_j(j, _):
            cseg = plsc.load_gather(cntbuf, [jnp.full((16,), j * 32 + wid,
                                                      jnp.int32)])
            trip = lax.div(jnp.max(jnp.minimum(cseg, CAP)) + 15, 16)
            bj = j * CAP

            def it(i, _):
                lane = i * 16 + iota
                pk = pkbuf[pl.ds(bj + i * 16, 16)]
                v = pvbuf[pl.ds(bj + i * 16, 16)]
                lm = lane < cseg
                spk, sv = plsc.sort_key_val(jnp.where(lm, pk, -1), v)
                am = spk >= 0
                floc = lax.shift_right_logical(spk, 18)
                flocc = jnp.clip(floc, 0, SHARD - 1)
                p = spk & 0x3FFFF
                g = plsc.load_gather(maxv_fb, [flocc])
                win = (am & (sv == g)).astype(jnp.int32)
                prev_ok = ((floc == jnp.take(floc, jnp.maximum(iota - 1, 0)))
                           & (iota >= 1)).astype(jnp.int32)
                cum = win
                c = prev_ok
                for d in (1, 2, 4, 8):
                    back = jnp.maximum(iota - d, 0)
                    cum = jnp.where(c != 0, cum | jnp.take(cum, back), cum)
                    c = c & jnp.take(c, back)
                prior = jnp.where(prev_ok != 0,
                                  jnp.take(cum, jnp.maximum(iota - 1, 0)), 0)
                first_win = (win != 0) & (prior == 0)
                ga = plsc.load_gather(argp_fb, [flocc])
                mm = first_win & (p < ga)
                plsc.store_scatter(argp_fb, [flocc], p, mask=mm)
                return 0

            lax.fori_loop(0, trip, it, 0)
            return 0

        lax.fori_loop(0, SEGB, pass2_j, 0)

        @pl.when(max_cnt > CAP)
        def _():
            def f2_chunk(ci, _):
                off = pt_base + ci * ACH
                pltpu.sync_copy(lin_hbm.at[pl.ds(off, ACH)], lin_v)
                pltpu.sync_copy(pvn_hbm.at[pl.ds(off, ACH)], pvn_v)

                def vloop(i, _):
                    l = lin_v[pl.ds(i * 16, 16)]
                    v = pvn_v[pl.ds(i * 16, 16)]
                    cs = lax.shift_right_logical(l, 4) & 31
                    act = (cs == wid) & (v > 0.0)
                    locs = (l & 15) * 512 + lax.shift_right_logical(l, 9)
                    p = ci * ACH + i * 16 + iota
                    g = plsc.load_gather(maxv_fb, [locs])
                    win = act & (v == g)
                    ga = plsc.load_gather(argp_fb, [locs])
                    need = win & (p < ga)

                    def body(m):
                        plsc.store_scatter(argp_fb, [locs], p, mask=m != 0)
                        ga2 = plsc.load_gather(argp_fb, [locs])
                        return (win & (p < ga2)).astype(jnp.int32)

                    lax.while_loop(lambda m: jnp.any(m != 0), body,
                                   need.astype(jnp.int32))
                    return 0

                lax.fori_loop(0, ACH // 16, vloop, 0)
                return 0

            lax.fori_loop(0, HW // ACH, f2_chunk, 0)

        def mkidx(i, _):
            s = i * 16 + iota
            y = s & 511
            sl = lax.shift_right_logical(s, 9)
            q = y * 512 + wid * 16 + sl
            a = argp_fb[pl.ds(i * 16, 16)]
            valid = (a < MAXP) & (q > 0)
            t = y * 16 + sl
            plsc.store_scatter(idx_v, [t],
                               jnp.where(valid, a, q) + (b * C) * HW)
            plsc.store_scatter(gath_v, [t],
                               jnp.where(valid, 1.0, 0.0))
            return 0

        lax.fori_loop(0, SHARD // 16, mkidx, 0)

        for c in range(C):
            if c > 0:
                def bump(i, _):
                    idx_v[pl.ds(i * 16, 16)] = idx_v[pl.ds(i * 16, 16)] + HW
                    return 0

                lax.fori_loop(0, SHARD // 16, bump, 0)
            pltpu.async_copy(x_hbm.at[idx_v], outb_v, sem).wait()

            def emit(i, _):
                t = i * 16 + iota
                vm = gath_v[pl.ds(i * 16, 16)] > 0.5
                gv = outb_v[pl.ds(i * 16, 16)]
                outb_v[pl.ds(i * 16, 16)] = jnp.where(
                    vm & (gv < 10000.0), gv, 0.0)
                return 0

            lax.fori_loop(0, SHARD // 16, emit, 0)
            pltpu.sync_copy(
                outb_v,
                out_hbm.at[pl.ds((wid * B * C + b * C + c) * SHARD, SHARD)])
        return 0

    lax.fori_loop(0, B, per_batch, 0)


@jax.jit
def kernel(x, flow_in):
    lin, pvn = _prep(x, flow_in)
    xf = x.reshape(B * C * HW)
    mesh = plsc.VectorSubcoreMesh(core_axis_name="c", subcore_axis_name="s")
    bink = functools.partial(
        pl.kernel,
        mesh=mesh,
        compiler_params=pltpu.CompilerParams(needs_layout_passes=False),
        out_type=(
            jax.ShapeDtypeStruct((NSEG * 32 * CAP,), jnp.int32),
            jax.ShapeDtypeStruct((NSEG * 32 * CAP,), jnp.float32),
            jax.ShapeDtypeStruct((NSEG * 32,), jnp.int32),
        ),
        scratch_types=[
            pltpu.VMEM((ACH,), jnp.int32),
            pltpu.VMEM((ACH,), jnp.float32),
            pltpu.VMEM((64,), jnp.int32),
            pltpu.VMEM((32 * CAP,), jnp.int32),
            pltpu.VMEM((32 * CAP,), jnp.float32),
            pltpu.SemaphoreType.DMA,
        ],
    )(_bin_body)
    pack_b, pvn_b, counts = bink(lin, pvn)

    splat = functools.partial(
        pl.kernel,
        mesh=mesh,
        compiler_params=pltpu.CompilerParams(needs_layout_passes=False),
        out_type=jax.ShapeDtypeStruct((NW * B * C * SHARD,), jnp.float32),
        scratch_types=[
            pltpu.VMEM((SEGB * CAP,), jnp.int32),
            pltpu.VMEM((SEGB * CAP,), jnp.float32),
            pltpu.VMEM((SEGB * 32,), jnp.int32),
            pltpu.VMEM((SHARD,), jnp.float32),
            pltpu.VMEM((SHARD,), jnp.int32),
            pltpu.VMEM((SHARD,), jnp.int32),
            pltpu.VMEM((SHARD,), jnp.float32),
            pltpu.VMEM((SHARD,), jnp.float32),
            pltpu.VMEM((ACH,), jnp.int32),
            pltpu.VMEM((ACH,), jnp.float32),
            pltpu.SemaphoreType.DMA,
            pltpu.SemaphoreType.DMA,
        ],
    )(_splat_body)
    out = splat(pack_b, pvn_b, counts, xf, lin, pvn)
    return out.reshape(NW, B, C, H, 16).transpose(1, 2, 3, 0, 4).reshape(B, C, H, W)


# speedup vs baseline: 1.0677x; 1.0677x over previous
import functools

import jax
import jax.numpy as jnp
from jax import lax
from jax.experimental import pallas as pl
from jax.experimental.pallas import tpu as pltpu
from jax.experimental.pallas import tpu_sc as plsc

B, C, H, W = 8, 3, 512, 512
HW = H * W
NW = 32
SHARD = HW // NW
NPC = 64
PCPTS = (B * HW) // NPC
PCB = NPC // B
ACH = 2048
NCH = PCPTS // ACH
CAP = 768
NSEG = NPC
SEGB = PCB
ROWS = 128
MAXP = HW


def _prep_body(flow_ref, depth_ref, lin_ref, pvn_ref):
    r = pl.program_id(1)
    fx = flow_ref[0, 0]
    fy = flow_ref[0, 1]
    gx = lax.broadcasted_iota(jnp.int32, (ROWS, W), 1).astype(jnp.float32)
    gy = lax.broadcasted_iota(jnp.int32, (ROWS, W), 0).astype(jnp.float32) \
        + (r * ROWS).astype(jnp.float32)
    cxf = jnp.round(gx + fx)
    cyf = jnp.round(gy + fy)
    inb = (cxf >= 0) & (cxf < W) & (cyf >= 0) & (cyf < H)
    cx = jnp.clip(cxf, 0, W - 1).astype(jnp.int32)
    cy = jnp.clip(cyf, 0, H - 1).astype(jnp.int32)
    lin_ref[0] = jnp.where(inb, cy * W + cx, 0)
    v = depth_ref[0, 0]
    pvn = 1.0 / (v + 1e-08)
    pvn_ref[0] = pvn * (pvn < 10000.0).astype(jnp.float32)


def _prep(x, flow_in):
    lin, pvn = pl.pallas_call(
        _prep_body,
        out_shape=(
            jax.ShapeDtypeStruct((B, H, W), jnp.int32),
            jax.ShapeDtypeStruct((B, H, W), jnp.float32),
        ),
        grid=(B, H // ROWS),
        in_specs=[
            pl.BlockSpec((1, 2, ROWS, W), lambda b, r: (b, 0, r, 0)),
            pl.BlockSpec((1, 1, ROWS, W), lambda b, r: (b, 2, r, 0)),
        ],
        out_specs=(
            pl.BlockSpec((1, ROWS, W), lambda b, r: (b, r, 0)),
            pl.BlockSpec((1, ROWS, W), lambda b, r: (b, r, 0)),
        ),
    )(flow_in, x)
    return lin.reshape(B * HW), pvn.reshape(B * HW)


def _bin_body(lin_hbm, pvn_hbm, pack_hbm, pvnb_hbm, cnt_hbm,
              lin_v, pvn_v, next_v, stage_pk, stage_pv, sem):
    wid = lax.axis_index("s") * 2 + lax.axis_index("c")
    iota = lax.iota(jnp.int32, 16)

    def per_pc(k, _):
        pc = wid * 2 + k
        pt0 = pc * PCPTS
        p_base = (pc % PCB) * PCPTS

        next_v[pl.ds(0, 16)] = jnp.zeros((16,), jnp.int32)
        next_v[pl.ds(16, 16)] = jnp.zeros((16,), jnp.int32)
        next_v[pl.ds(32, 16)] = jnp.zeros((16,), jnp.int32)
        next_v[pl.ds(48, 16)] = jnp.zeros((16,), jnp.int32)

        def per_chunk(ch, _):
            off = pt0 + ch * ACH
            pltpu.sync_copy(lin_hbm.at[pl.ds(off, ACH)], lin_v)
            pltpu.sync_copy(pvn_hbm.at[pl.ds(off, ACH)], pvn_v)

            def vloop(i, _):
                l = lin_v[pl.ds(i * 16, 16)]
                v = pvn_v[pl.ds(i * 16, 16)]
                act = v > 0.0
                bkt = lax.shift_right_logical(l, 4) & 31
                key = jnp.where(act, bkt, 63)
                p = p_base + ch * ACH + i * 16 + iota
                floc = (l & 15) * 512 + lax.shift_right_logical(l, 9)
                pack = lax.shift_left(floc, 18) | p
                skey, sval = plsc.sort_key_val(key, iota)
                v_s = jnp.take(v, sval)
                pack_s = jnp.take(pack, sval)
                act_s = skey < 32
                st = iota
                c = ((skey == jnp.take(skey, jnp.maximum(iota - 1, 0)))
                     & (iota >= 1)).astype(jnp.int32)
                for d in (1, 2, 4, 8):
                    back = jnp.maximum(iota - d, 0)
                    st = jnp.where(c != 0, jnp.take(st, back), st)
                    c = c & jnp.take(c, back)
                rank = iota - st
                nxt_key = jnp.take(skey, jnp.minimum(iota + 1, 15))
                is_last = (iota == 15) | (nxt_key != skey)
                cur = plsc.load_gather(next_v, [skey])
                pos = cur + rank
                valid = act_s & (pos < CAP)
                plsc.store_scatter(next_v, [skey], pos + 1,
                                   mask=is_last & act_s)
                sidx = skey * CAP + jnp.minimum(pos, CAP - 1)
                plsc.store_scatter(stage_pk, [sidx], pack_s, mask=valid)
                plsc.store_scatter(stage_pv, [sidx], v_s, mask=valid)
                return 0

            lax.fori_loop(0, ACH // 16, vloop, 0)
            return 0

        lax.fori_loop(0, NCH, per_chunk, 0)
        pltpu.sync_copy(stage_pk,
                        pack_hbm.at[pl.ds(pc * 32 * CAP, 32 * CAP)])
        pltpu.sync_copy(stage_pv,
                        pvnb_hbm.at[pl.ds(pc * 32 * CAP, 32 * CAP)])
        pltpu.sync_copy(next_v.at[pl.ds(0, 32)],
                        cnt_hbm.at[pl.ds(pc * 32, 32)])
        return 0

    lax.fori_loop(0, 2, per_pc, 0)


def _splat_body(pack_hbm, pvnb_hbm, cnt_hbm, x_hbm, lin_hbm, pvn_hbm, out_hbm,
                pkbuf, pvbuf, cntbuf, maxv_fb, argp_fb, idx_v, idxb_v, idxc_v,
                gath_v, gathb_v, gathc_v, flag_v, outb_v,
                lin_v, pvn_v, sem, sem2, sem3):
    wid = lax.axis_index("s") * 2 + lax.axis_index("c")
    iota = lax.iota(jnp.int32, 16)

    def per_batch(b, _):
        pt_base = b * HW

        def init(i, _):
            maxv_fb[pl.ds(i * 16, 16)] = jnp.zeros((16,), jnp.float32)
            argp_fb[pl.ds(i * 16, 16)] = jnp.full((16,), MAXP, jnp.int32)
            return 0

        lax.fori_loop(0, SHARD // 16, init, 0)

        pltpu.sync_copy(cnt_hbm.at[pl.ds(b * SEGB * 32, SEGB * 32)], cntbuf)

        def fire(j, _):
            soff = ((b * SEGB + j) * 32 + wid) * CAP
            pltpu.async_copy(pack_hbm.at[pl.ds(soff, CAP)],
                             pkbuf.at[pl.ds(j * CAP, CAP)], sem)
            pltpu.async_copy(pvnb_hbm.at[pl.ds(soff, CAP)],
                             pvbuf.at[pl.ds(j * CAP, CAP)], sem2)
            return 0

        lax.fori_loop(0, SEGB, fire, 0)
        pltpu.make_async_copy(pack_hbm.at[pl.ds(0, SEGB * CAP)], pkbuf, sem).wait()
        pltpu.make_async_copy(pvnb_hbm.at[pl.ds(0, SEGB * CAP)], pvbuf, sem2).wait()

        def ovf_scan(j, m):
            cj = cntbuf[pl.ds(j * 16, 16)]
            return jnp.maximum(m, jnp.max(cj))

        max_cnt = lax.fori_loop(0, (SEGB * 32) // 16, ovf_scan, jnp.int32(0))

        def pass1_j(j, _):
            cseg = plsc.load_gather(cntbuf, [jnp.full((16,), j * 32 + wid,
                                                      jnp.int32)])
            trip = lax.div(jnp.max(jnp.minimum(cseg, CAP)) + 15, 16)
            bj = j * CAP

            def it(i, _):
                lane = i * 16 + iota
                pk = pkbuf[pl.ds(bj + i * 16, 16)]
                v = pvbuf[pl.ds(bj + i * 16, 16)]
                lm = lane < cseg
                spk, sv = plsc.sort_key_val(jnp.where(lm, pk, -1), v)
                am = spk >= 0
                floc = lax.shift_right_logical(spk, 18)
                prev = jnp.take(floc, jnp.maximum(iota - 1, 0))
                m = sv
                c = ((floc == prev) & (iota >= 1)).astype(jnp.int32)
                for d in (1, 2, 4, 8):
                    back = jnp.maximum(iota - d, 0)
                    m = jnp.where(c != 0, jnp.maximum(m, jnp.take(m, back)), m)
                    c = c & jnp.take(c, back)
                nxt = jnp.take(floc, jnp.minimum(iota + 1, 15))
                is_last = (iota == 15) | (nxt != floc)
                flocc = jnp.clip(floc, 0, SHARD - 1)
                g = plsc.load_gather(maxv_fb, [flocc])
                mm = am & is_last & (m > g)
                plsc.store_scatter(maxv_fb, [flocc], m, mask=mm)
                return 0

            lax.fori_loop(0, trip, it, 0)
            return 0

        lax.fori_loop(0, SEGB, pass1_j, 0)

        @pl.when(max_cnt > CAP)
        def _():
            def f1_chunk(ci, _):
                off = pt_base + ci * ACH
                pltpu.sync_copy(lin_hbm.at[pl.ds(off, ACH)], lin_v)
                pltpu.sync_copy(pvn_hbm.at[pl.ds(off, ACH)], pvn_v)

                def vloop(i, _):
                    l = lin_v[pl.ds(i * 16, 16)]
                    v = pvn_v[pl.ds(i * 16, 16)]
                    cs = lax.shift_right_logical(l, 4) & 31
                    act = (cs == wid) & (v > 0.0)
                    locs = (l & 15) * 512 + lax.shift_right_logical(l, 9)
                    g = plsc.load_gather(maxv_fb, [locs])
                    need = act & (v > g)

                    def body(m):
                        plsc.store_scatter(maxv_fb, [locs], v, mask=m != 0)
                        g2 = plsc.load_gather(maxv_fb, [locs])
                        return (act & (v > g2)).astype(jnp.int32)

                    lax.while_loop(lambda m: jnp.any(m != 0), body,
                                   need.astype(jnp.int32))
                    return 0

                lax.fori_loop(0, ACH // 16, vloop, 0)
                return 0

            lax.fori_loop(0, HW // ACH, f1_chunk, 0)

        def pass2_j(j, _):
            cseg = plsc.load_gather(cntbuf, [jnp.full((16,), j * 32 + wid,
                                                      jnp.int32)])
            trip = lax.div(jnp.max(jnp.minimum(cseg, CAP)) + 15, 16)
            bj = j * CAP

            def it(i, _):
                lane = i * 16 + iota
                pk = pkbuf[pl.ds(bj + i * 16, 16)]
                v = pvbuf[pl.ds(bj + i * 16, 16)]
                lm = lane < cseg
                spk, sv = plsc.sort_key_val(jnp.where(lm, pk, -1), v)
                am = spk >= 0
                floc = lax.shift_right_logical(spk, 18)
                flocc = jnp.clip(floc, 0, SHARD - 1)
                p = spk & 0x3FFFF
                g = plsc.load_gather(maxv_fb, [flocc])
                win = (am & (sv == g)).astype(jnp.int32)
                prev_ok = ((floc == jnp.take(floc, jnp.maximum(iota - 1, 0)))
                           & (iota >= 1)).astype(jnp.int32)
                cum = win
                c = prev_ok
                for d in (1, 2, 4, 8):
                    back = jnp.maximum(iota - d, 0)
                    cum = jnp.where(c != 0, cum | jnp.take(cum, back), cum)
                    c = c & jnp.take(c, back)
                prior = jnp.where(prev_ok != 0,
                                  jnp.take(cum, jnp.maximum(iota - 1, 0)), 0)
                first_win = (win != 0) & (prior == 0)
                ga = plsc.load_gather(argp_fb, [flocc])
                mm = first_win & (p < ga)
                plsc.store_scatter(argp_fb, [flocc], p, mask=mm)
                return 0

            lax.fori_loop(0, trip, it, 0)
            return 0

        lax.fori_loop(0, SEGB, pass2_j, 0)

        @pl.when(max_cnt > CAP)
        def _():
            def f2_chunk(ci, _):
                off = pt_base + ci * ACH
                pltpu.sync_copy(lin_hbm.at[pl.ds(off, ACH)], lin_v)
                pltpu.sync_copy(pvn_hbm.at[pl.ds(off, ACH)], pvn_v)

                def vloop(i, _):
                    l = lin_v[pl.ds(i * 16, 16)]
                    v = pvn_v[pl.ds(i * 16, 16)]
                    cs = lax.shift_right_logical(l, 4) & 31
                    act = (cs == wid) & (v > 0.0)
                    locs = (l & 15) * 512 + lax.shift_right_logical(l, 9)
                    p = ci * ACH + i * 16 + iota
                    g = plsc.load_gather(maxv_fb, [locs])
                    win = act & (v == g)
                    ga = plsc.load_gather(argp_fb, [locs])
                    need = win & (p < ga)

                    def body(m):
                        plsc.store_scatter(argp_fb, [locs], p, mask=m != 0)
                        ga2 = plsc.load_gather(argp_fb, [locs])
                        return (win & (p < ga2)).astype(jnp.int32)

                    lax.while_loop(lambda m: jnp.any(m != 0), body,
                                   need.astype(jnp.int32))
                    return 0

                lax.fori_loop(0, ACH // 16, vloop, 0)
                return 0

            lax.fori_loop(0, HW // ACH, f2_chunk, 0)

        def mkidx(i, _):
            s = i * 16 + iota
            y = s & 511
            sl = lax.shift_right_logical(s, 9)
            q = y * 512 + wid * 16 + sl
            a = argp_fb[pl.ds(i * 16, 16)]
            valid = (a < MAXP) & (q > 0)
            t = y * 16 + sl
            base_i = jnp.where(valid, a, q) + (b * C) * HW
            plsc.store_scatter(idx_v, [t], base_i)
            plsc.store_scatter(idxb_v, [t], base_i + HW)
            plsc.store_scatter(idxc_v, [t], base_i + 2 * HW)
            plsc.store_scatter(flag_v, [t], valid.astype(jnp.int32))
            return 0

        lax.fori_loop(0, SHARD // 16, mkidx, 0)
        handles = [
            pltpu.async_copy(x_hbm.at[idx_v], gath_v, sem),
            pltpu.async_copy(x_hbm.at[idxb_v], gathb_v, sem2),
            pltpu.async_copy(x_hbm.at[idxc_v], gathc_v, sem3),
        ]
        for c, gbuf in enumerate((gath_v, gathb_v, gathc_v)):
            handles[c].wait()

            def emit(i, _):
                vm = flag_v[pl.ds(i * 16, 16)] != 0
                gv = gbuf[pl.ds(i * 16, 16)]
                outb_v[pl.ds(i * 16, 16)] = jnp.where(
                    vm & (gv < 10000.0), gv, 0.0)
                return 0

            lax.fori_loop(0, SHARD // 16, emit, 0)
            pltpu.sync_copy(
                outb_v,
                out_hbm.at[pl.ds((wid * B * C + b * C + c) * SHARD, SHARD)])
        return 0

    lax.fori_loop(0, B, per_batch, 0)


@jax.jit
def kernel(x, flow_in):
    lin, pvn = _prep(x, flow_in)
    xf = x.reshape(B * C * HW)
    mesh = plsc.VectorSubcoreMesh(core_axis_name="c", subcore_axis_name="s")
    bink = functools.partial(
        pl.kernel,
        mesh=mesh,
        compiler_params=pltpu.CompilerParams(needs_layout_passes=False),
        out_type=(
            jax.ShapeDtypeStruct((NSEG * 32 * CAP,), jnp.int32),
            jax.ShapeDtypeStruct((NSEG * 32 * CAP,), jnp.float32),
            jax.ShapeDtypeStruct((NSEG * 32,), jnp.int32),
        ),
        scratch_types=[
            pltpu.VMEM((ACH,), jnp.int32),
            pltpu.VMEM((ACH,), jnp.float32),
            pltpu.VMEM((64,), jnp.int32),
            pltpu.VMEM((32 * CAP,), jnp.int32),
            pltpu.VMEM((32 * CAP,), jnp.float32),
            pltpu.SemaphoreType.DMA,
        ],
    )(_bin_body)
    pack_b, pvn_b, counts = bink(lin, pvn)

    splat = functools.partial(
        pl.kernel,
        mesh=mesh,
        compiler_params=pltpu.CompilerParams(needs_layout_passes=False),
        out_type=jax.ShapeDtypeStruct((NW * B * C * SHARD,), jnp.float32),
        scratch_types=[
            pltpu.VMEM((SEGB * CAP,), jnp.int32),
            pltpu.VMEM((SEGB * CAP,), jnp.float32),
            pltpu.VMEM((SEGB * 32,), jnp.int32),
            pltpu.VMEM((SHARD,), jnp.float32),
            pltpu.VMEM((SHARD,), jnp.int32),
            pltpu.VMEM((SHARD,), jnp.int32),
            pltpu.VMEM((SHARD,), jnp.int32),
            pltpu.VMEM((SHARD,), jnp.int32),
            pltpu.VMEM((SHARD,), jnp.float32),
            pltpu.VMEM((SHARD,), jnp.float32),
            pltpu.VMEM((SHARD,), jnp.float32),
            pltpu.VMEM((SHARD,), jnp.int32),
            pltpu.VMEM((SHARD,), jnp.float32),
            pltpu.VMEM((ACH,), jnp.int32),
            pltpu.VMEM((ACH,), jnp.float32),
            pltpu.SemaphoreType.DMA,
            pltpu.SemaphoreType.DMA,
            pltpu.SemaphoreType.DMA,
        ],
    )(_splat_body)
    out = splat(pack_b, pvn_b, counts, xf, lin, pvn)
    return out.reshape(NW, B, C, H, 16).transpose(1, 2, 3, 0, 4).reshape(B, C, H, W)


# speedup vs baseline: 1.0867x; 1.0178x over previous
import functools

import jax
import jax.numpy as jnp
from jax import lax
from jax.experimental import pallas as pl
from jax.experimental.pallas import tpu as pltpu
from jax.experimental.pallas import tpu_sc as plsc

B, C, H, W = 8, 3, 512, 512
HW = H * W
NW = 32
SHARD = HW // NW
NPC = 64
PCPTS = (B * HW) // NPC
PCB = NPC // B
ACH = 4096
NCH = PCPTS // ACH
CAP = 768
NSEG = NPC
SEGB = PCB
ROWS = 128
MAXP = HW


def _prep_body(flow_ref, depth_ref, lin_ref, pvn_ref):
    r = pl.program_id(1)
    fx = flow_ref[0, 0]
    fy = flow_ref[0, 1]
    gx = lax.broadcasted_iota(jnp.int32, (ROWS, W), 1).astype(jnp.float32)
    gy = lax.broadcasted_iota(jnp.int32, (ROWS, W), 0).astype(jnp.float32) \
        + (r * ROWS).astype(jnp.float32)
    cxf = jnp.round(gx + fx)
    cyf = jnp.round(gy + fy)
    inb = (cxf >= 0) & (cxf < W) & (cyf >= 0) & (cyf < H)
    cx = jnp.clip(cxf, 0, W - 1).astype(jnp.int32)
    cy = jnp.clip(cyf, 0, H - 1).astype(jnp.int32)
    lin_ref[0] = jnp.where(inb, cy * W + cx, 0)
    v = depth_ref[0, 0]
    pvn = 1.0 / (v + 1e-08)
    pvn_ref[0] = pvn * (pvn < 10000.0).astype(jnp.float32)


def _prep(x, flow_in):
    lin, pvn = pl.pallas_call(
        _prep_body,
        out_shape=(
            jax.ShapeDtypeStruct((B, H, W), jnp.int32),
            jax.ShapeDtypeStruct((B, H, W), jnp.float32),
        ),
        grid=(B, H // ROWS),
        in_specs=[
            pl.BlockSpec((1, 2, ROWS, W), lambda b, r: (b, 0, r, 0)),
            pl.BlockSpec((1, 1, ROWS, W), lambda b, r: (b, 2, r, 0)),
        ],
        out_specs=(
            pl.BlockSpec((1, ROWS, W), lambda b, r: (b, r, 0)),
            pl.BlockSpec((1, ROWS, W), lambda b, r: (b, r, 0)),
        ),
    )(flow_in, x)
    return lin.reshape(B * HW), pvn.reshape(B * HW)


def _bin_body(lin_hbm, pvn_hbm, pack_hbm, pvnb_hbm, cnt_hbm,
              lin_v, pvn_v, next_v, stage_pk, stage_pv, sem):
    wid = lax.axis_index("s") * 2 + lax.axis_index("c")
    iota = lax.iota(jnp.int32, 16)

    def per_pc(k, _):
        pc = wid * 2 + k
        pt0 = pc * PCPTS
        p_base = (pc % PCB) * PCPTS

        next_v[pl.ds(0, 16)] = jnp.zeros((16,), jnp.int32)
        next_v[pl.ds(16, 16)] = jnp.zeros((16,), jnp.int32)
        next_v[pl.ds(32, 16)] = jnp.zeros((16,), jnp.int32)
        next_v[pl.ds(48, 16)] = jnp.zeros((16,), jnp.int32)

        def per_chunk(ch, _):
            off = pt0 + ch * ACH
            pltpu.sync_copy(lin_hbm.at[pl.ds(off, ACH)], lin_v)
            pltpu.sync_copy(pvn_hbm.at[pl.ds(off, ACH)], pvn_v)

            def vloop(i, _):
                l = lin_v[pl.ds(i * 16, 16)]
                v = pvn_v[pl.ds(i * 16, 16)]
                act = v > 0.0
                bkt = lax.shift_right_logical(l, 4) & 31
                key = jnp.where(act, bkt, 63)
                p = p_base + ch * ACH + i * 16 + iota
                floc = (l & 15) * 512 + lax.shift_right_logical(l, 9)
                pack = lax.shift_left(floc, 18) | p
                skey, sval = plsc.sort_key_val(key, iota)
                v_s = jnp.take(v, sval)
                pack_s = jnp.take(pack, sval)
                act_s = skey < 32
                st = iota
                c = ((skey == jnp.take(skey, jnp.maximum(iota - 1, 0)))
                     & (iota >= 1)).astype(jnp.int32)
                for d in (1, 2, 4, 8):
                    back = jnp.maximum(iota - d, 0)
                    st = jnp.where(c != 0, jnp.take(st, back), st)
                    c = c & jnp.take(c, back)
                rank = iota - st
                nxt_key = jnp.take(skey, jnp.minimum(iota + 1, 15))
                is_last = (iota == 15) | (nxt_key != skey)
                cur = plsc.load_gather(next_v, [skey])
                pos = cur + rank
                valid = act_s & (pos < CAP)
                plsc.store_scatter(next_v, [skey], pos + 1,
                                   mask=is_last & act_s)
                sidx = skey * CAP + jnp.minimum(pos, CAP - 1)
                plsc.store_scatter(stage_pk, [sidx], pack_s, mask=valid)
                plsc.store_scatter(stage_pv, [sidx], v_s, mask=valid)
                return 0

            lax.fori_loop(0, ACH // 16, vloop, 0)
            return 0

        lax.fori_loop(0, NCH, per_chunk, 0)
        pltpu.sync_copy(stage_pk,
                        pack_hbm.at[pl.ds(pc * 32 * CAP, 32 * CAP)])
        pltpu.sync_copy(stage_pv,
                        pvnb_hbm.at[pl.ds(pc * 32 * CAP, 32 * CAP)])
        pltpu.sync_copy(next_v.at[pl.ds(0, 32)],
                        cnt_hbm.at[pl.ds(pc * 32, 32)])
        return 0

    lax.fori_loop(0, 2, per_pc, 0)


def _splat_body(pack_hbm, pvnb_hbm, cnt_hbm, x_hbm, lin_hbm, pvn_hbm, out_hbm,
                pkbuf, pvbuf, cntbuf, maxv_fb, argp_fb, idx_v, idxb_v, idxc_v,
                gath_v, gathb_v, gathc_v, flag_v, outb_v,
                lin_v, pvn_v, sem, sem2, sem3):
    wid = lax.axis_index("s") * 2 + lax.axis_index("c")
    iota = lax.iota(jnp.int32, 16)

    def per_batch(b, _):
        pt_base = b * HW

        def init(i, _):
            maxv_fb[pl.ds(i * 16, 16)] = jnp.zeros((16,), jnp.float32)
            argp_fb[pl.ds(i * 16, 16)] = jnp.full((16,), MAXP, jnp.int32)
            return 0

        lax.fori_loop(0, SHARD // 16, init, 0)

        pltpu.sync_copy(cnt_hbm.at[pl.ds(b * SEGB * 32, SEGB * 32)], cntbuf)

        def fire(j, _):
            soff = ((b * SEGB + j) * 32 + wid) * CAP
            pltpu.async_copy(pack_hbm.at[pl.ds(soff, CAP)],
                             pkbuf.at[pl.ds(j * CAP, CAP)], sem)
            pltpu.async_copy(pvnb_hbm.at[pl.ds(soff, CAP)],
                             pvbuf.at[pl.ds(j * CAP, CAP)], sem2)
            return 0

        lax.fori_loop(0, SEGB, fire, 0)
        pltpu.make_async_copy(pack_hbm.at[pl.ds(0, SEGB * CAP)], pkbuf, sem).wait()
        pltpu.make_async_copy(pvnb_hbm.at[pl.ds(0, SEGB * CAP)], pvbuf, sem2).wait()

        def ovf_scan(j, m):
            cj = cntbuf[pl.ds(j * 16, 16)]
            return jnp.maximum(m, jnp.max(cj))

        max_cnt = lax.fori_loop(0, (SEGB * 32) // 16, ovf_scan, jnp.int32(0))

        def pass1_j(j, _):
            cseg = plsc.load_gather(cntbuf, [jnp.full((16,), j * 32 + wid,
                                                      jnp.int32)])
            trip = lax.div(jnp.max(jnp.minimum(cseg, CAP)) + 15, 16)
            bj = j * CAP

            def it(i, _):
                lane = i * 16 + iota
                pk = pkbuf[pl.ds(bj + i * 16, 16)]
                v = pvbuf[pl.ds(bj + i * 16, 16)]
                lm = lane < cseg
                spk, sv = plsc.sort_key_val(jnp.where(lm, pk, -1), v)
                am = spk >= 0
                floc = lax.shift_right_logical(spk, 18)
                prev = jnp.take(floc, jnp.maximum(iota - 1, 0))
                m = sv
                c = ((floc == prev) & (iota >= 1)).astype(jnp.int32)
                for d in (1, 2, 4, 8):
                    back = jnp.maximum(iota - d, 0)
                    m = jnp.where(c != 0, jnp.maximum(m, jnp.take(m, back)), m)
                    c = c & jnp.take(c, back)
                nxt = jnp.take(floc, jnp.minimum(iota + 1, 15))
                is_last = (iota == 15) | (nxt != floc)
                flocc = jnp.clip(floc, 0, SHARD - 1)
                g = plsc.load_gather(maxv_fb, [flocc])
                mm = am & is_last & (m > g)
                plsc.store_scatter(maxv_fb, [flocc], m, mask=mm)
                return 0

            lax.fori_loop(0, trip, it, 0)
            return 0

        lax.fori_loop(0, SEGB, pass1_j, 0)

        @pl.when(max_cnt > CAP)
        def _():
            def f1_chunk(ci, _):
                off = pt_base + ci * ACH
                pltpu.sync_copy(lin_hbm.at[pl.ds(off, ACH)], lin_v)
                pltpu.sync_copy(pvn_hbm.at[pl.ds(off, ACH)], pvn_v)

                def vloop(i, _):
                    l = lin_v[pl.ds(i * 16, 16)]
                    v = pvn_v[pl.ds(i * 16, 16)]
                    cs = lax.shift_right_logical(l, 4) & 31
                    act = (cs == wid) & (v > 0.0)
                    locs = (l & 15) * 512 + lax.shift_right_logical(l, 9)
                    g = plsc.load_gather(maxv_fb, [locs])
                    need = act & (v > g)

                    def body(m):
                        plsc.store_scatter(maxv_fb, [locs], v, mask=m != 0)
                        g2 = plsc.load_gather(maxv_fb, [locs])
                        return (act & (v > g2)).astype(jnp.int32)

                    lax.while_loop(lambda m: jnp.any(m != 0), body,
                                   need.astype(jnp.int32))
                    return 0

                lax.fori_loop(0, ACH // 16, vloop, 0)
                return 0

            lax.fori_loop(0, HW // ACH, f1_chunk, 0)

        def pass2_j(j, _):
            cseg = plsc.load_gather(cntbuf, [jnp.full((16,), j * 32 + wid,
                                                      jnp.int32)])
            trip = lax.div(jnp.max(jnp.minimum(cseg, CAP)) + 15, 16)
            bj = j * CAP

            def it(i, _):
                lane = i * 16 + iota
                pk = pkbuf[pl.ds(bj + i * 16, 16)]
                v = pvbuf[pl.ds(bj + i * 16, 16)]
                lm = lane < cseg
                spk, sv = plsc.sort_key_val(jnp.where(lm, pk, -1), v)
                am = spk >= 0
                floc = lax.shift_right_logical(spk, 18)
                flocc = jnp.clip(floc, 0, SHARD - 1)
                p = spk & 0x3FFFF
                g = plsc.load_gather(maxv_fb, [flocc])
                win = (am & (sv == g)).astype(jnp.int32)
                prev_ok = ((floc == jnp.take(floc, jnp.maximum(iota - 1, 0)))
                           & (iota >= 1)).astype(jnp.int32)
                cum = win
                c = prev_ok
                for d in (1, 2, 4, 8):
                    back = jnp.maximum(iota - d, 0)
                    cum = jnp.where(c != 0, cum | jnp.take(cum, back), cum)
                    c = c & jnp.take(c, back)
                prior = jnp.where(prev_ok != 0,
                                  jnp.take(cum, jnp.maximum(iota - 1, 0)), 0)
                first_win = (win != 0) & (prior == 0)
                ga = plsc.load_gather(argp_fb, [flocc])
                mm = first_win & (p < ga)
                plsc.store_scatter(argp_fb, [flocc], p, mask=mm)
                return 0

            lax.fori_loop(0, trip, it, 0)
            return 0

        lax.fori_loop(0, SEGB, pass2_j, 0)

        @pl.when(max_cnt > CAP)
        def _():
            def f2_chunk(ci, _):
                off = pt_base + ci * ACH
                pltpu.sync_copy(lin_hbm.at[pl.ds(off, ACH)], lin_v)
                pltpu.sync_copy(pvn_hbm.at[pl.ds(off, ACH)], pvn_v)

                def vloop(i, _):
                    l = lin_v[pl.ds(i * 16, 16)]
                    v = pvn_v[pl.ds(i * 16, 16)]
                    cs = lax.shift_right_logical(l, 4) & 31
                    act = (cs == wid) & (v > 0.0)
                    locs = (l & 15) * 512 + lax.shift_right_logical(l, 9)
                    p = ci * ACH + i * 16 + iota
                    g = plsc.load_gather(maxv_fb, [locs])
                    win = act & (v == g)
                    ga = plsc.load_gather(argp_fb, [locs])
                    need = win & (p < ga)

                    def body(m):
                        plsc.store_scatter(argp_fb, [locs], p, mask=m != 0)
                        ga2 = plsc.load_gather(argp_fb, [locs])
                        return (win & (p < ga2)).astype(jnp.int32)

                    lax.while_loop(lambda m: jnp.any(m != 0), body,
                                   need.astype(jnp.int32))
                    return 0

                lax.fori_loop(0, ACH // 16, vloop, 0)
                return 0

            lax.fori_loop(0, HW // ACH, f2_chunk, 0)

        def mkidx(i, _):
            s = i * 16 + iota
            y = s & 511
            sl = lax.shift_right_logical(s, 9)
            q = y * 512 + wid * 16 + sl
            a = argp_fb[pl.ds(i * 16, 16)]
            valid = (a < MAXP) & (q > 0)
            t = y * 16 + sl
            base_i = jnp.where(valid, a, q) + (b * C) * HW
            plsc.store_scatter(idx_v, [t], base_i)
            plsc.store_scatter(idxb_v, [t], base_i + HW)
            plsc.store_scatter(idxc_v, [t], base_i + 2 * HW)
            plsc.store_scatter(flag_v, [t], valid.astype(jnp.int32))
            return 0

        lax.fori_loop(0, SHARD // 16, mkidx, 0)
        handles = [
            pltpu.async_copy(x_hbm.at[idx_v], gath_v, sem),
            pltpu.async_copy(x_hbm.at[idxb_v], gathb_v, sem2),
            pltpu.async_copy(x_hbm.at[idxc_v], gathc_v, sem3),
        ]
        for c, gbuf in enumerate((gath_v, gathb_v, gathc_v)):
            handles[c].wait()

            def emit(i, _):
                vm = flag_v[pl.ds(i * 16, 16)] != 0
                gv = gbuf[pl.ds(i * 16, 16)]
                outb_v[pl.ds(i * 16, 16)] = jnp.where(
                    vm & (gv < 10000.0), gv, 0.0)
                return 0

            lax.fori_loop(0, SHARD // 16, emit, 0)
            pltpu.sync_copy(
                outb_v,
                out_hbm.at[pl.ds((wid * B * C + b * C + c) * SHARD, SHARD)])
        return 0

    lax.fori_loop(0, B, per_batch, 0)


@jax.jit
def kernel(x, flow_in):
    lin, pvn = _prep(x, flow_in)
    xf = x.reshape(B * C * HW)
    mesh = plsc.VectorSubcoreMesh(core_axis_name="c", subcore_axis_name="s")
    bink = functools.partial(
        pl.kernel,
        mesh=mesh,
        compiler_params=pltpu.CompilerParams(needs_layout_passes=False),
        out_type=(
            jax.ShapeDtypeStruct((NSEG * 32 * CAP,), jnp.int32),
            jax.ShapeDtypeStruct((NSEG * 32 * CAP,), jnp.float32),
            jax.ShapeDtypeStruct((NSEG * 32,), jnp.int32),
        ),
        scratch_types=[
            pltpu.VMEM((ACH,), jnp.int32),
            pltpu.VMEM((ACH,), jnp.float32),
            pltpu.VMEM((64,), jnp.int32),
            pltpu.VMEM((32 * CAP,), jnp.int32),
            pltpu.VMEM((32 * CAP,), jnp.float32),
            pltpu.SemaphoreType.DMA,
        ],
    )(_bin_body)
    pack_b, pvn_b, counts = bink(lin, pvn)

    splat = functools.partial(
        pl.kernel,
        mesh=mesh,
        compiler_params=pltpu.CompilerParams(needs_layout_passes=False),
        out_type=jax.ShapeDtypeStruct((NW * B * C * SHARD,), jnp.float32),
        scratch_types=[
            pltpu.VMEM((SEGB * CAP,), jnp.int32),
            pltpu.VMEM((SEGB * CAP,), jnp.float32),
            pltpu.VMEM((SEGB * 32,), jnp.int32),
            pltpu.VMEM((SHARD,), jnp.float32),
            pltpu.VMEM((SHARD,), jnp.int32),
            pltpu.VMEM((SHARD,), jnp.int32),
            pltpu.VMEM((SHARD,), jnp.int32),
            pltpu.VMEM((SHARD,), jnp.int32),
            pltpu.VMEM((SHARD,), jnp.float32),
            pltpu.VMEM((SHARD,), jnp.float32),
            pltpu.VMEM((SHARD,), jnp.float32),
            pltpu.VMEM((SHARD,), jnp.int32),
            pltpu.VMEM((SHARD,), jnp.float32),
            pltpu.VMEM((ACH,), jnp.int32),
            pltpu.VMEM((ACH,), jnp.float32),
            pltpu.SemaphoreType.DMA,
            pltpu.SemaphoreType.DMA,
            pltpu.SemaphoreType.DMA,
        ],
    )(_splat_body)
    out = splat(pack_b, pvn_b, counts, xf, lin, pvn)
    return out.reshape(NW, B, C, H, 16).transpose(1, 2, 3, 0, 4).reshape(B, C, H, W)
